# Initial kernel scaffold; baseline (speedup 1.0000x reference)
import jax
import jax.numpy as jnp
from jax.experimental import pallas as pl


def kernel(hidden_states, weight):
    raise NotImplementedError("write your pallas kernel here")


# speedup vs baseline: 1.7617x; 1.7617x over previous
import functools

import jax
import jax.numpy as jnp
from jax.experimental import pallas as pl
from jax.experimental.pallas import tpu as pltpu

TOPK_EPS = 1e-20
BLOCK_M = 1024


def _gate_block(x_ref, wt_ref, idx_ref, w_ref):
    x = x_ref[...]
    wt = wt_ref[...]
    logits = jnp.dot(x, wt, preferred_element_type=jnp.float32)
    m, e = logits.shape
    iota = jax.lax.broadcasted_iota(jnp.int32, (m, e), 1)

    m1 = jnp.max(logits, axis=-1, keepdims=True)
    idx1 = jnp.min(jnp.where(logits == m1, iota, e), axis=-1, keepdims=True)
    masked = jnp.where(iota == idx1, -jnp.inf, logits)
    m2 = jnp.max(masked, axis=-1, keepdims=True)
    idx2 = jnp.min(jnp.where(masked == m2, iota, e), axis=-1, keepdims=True)

    z = jnp.sum(jnp.exp(logits - m1), axis=-1, keepdims=True)
    s1 = 1.0 / z
    s2 = jnp.exp(m2 - m1) / z
    denom = s1 + s2 + TOPK_EPS
    idx_ref[...] = jnp.concatenate([idx1, idx2], axis=1)
    w_ref[...] = jnp.concatenate([s1 / denom, s2 / denom], axis=1)


@functools.partial(jax.jit, static_argnames=("interpret",))
def _gate(x, wt, interpret=False):
    n, h = x.shape
    e = wt.shape[1]
    grid = (n // BLOCK_M,)
    return pl.pallas_call(
        _gate_block,
        grid=grid,
        in_specs=[
            pl.BlockSpec((BLOCK_M, h), lambda i: (i, 0)),
            pl.BlockSpec((h, e), lambda i: (0, 0)),
        ],
        out_specs=[
            pl.BlockSpec((BLOCK_M, 2), lambda i: (i, 0)),
            pl.BlockSpec((BLOCK_M, 2), lambda i: (i, 0)),
        ],
        out_shape=[
            jax.ShapeDtypeStruct((n, 2), jnp.int32),
            jax.ShapeDtypeStruct((n, 2), jnp.float32),
        ],
        interpret=interpret,
    )(x, wt)


def kernel(hidden_states, weight):
    bsz, seq_len, h = hidden_states.shape
    x = hidden_states.reshape(-1, h)
    topk_idx, topk_weight = _gate(x, weight.T)
    return (
        topk_idx.reshape(bsz, seq_len, -1),
        topk_weight.reshape(bsz, seq_len, -1),
    )


# speedup vs baseline: 1.8371x; 1.0428x over previous
import functools

import jax
import jax.numpy as jnp
from jax.experimental import pallas as pl
from jax.experimental.pallas import tpu as pltpu

TOPK_EPS = 1e-20
BLOCK_M = 2048


def _gate_block(x_ref, wt_ref, idx_ref, w_ref):
    x = x_ref[...]
    wt = wt_ref[...]
    logits = jnp.dot(x, wt, preferred_element_type=jnp.float32)
    m, e = logits.shape
    iota = jax.lax.broadcasted_iota(jnp.int32, (m, e), 1)

    m1 = jnp.max(logits, axis=-1, keepdims=True)
    idx1 = jnp.min(jnp.where(logits == m1, iota, e), axis=-1, keepdims=True)
    masked = jnp.where(iota == idx1, -jnp.inf, logits)
    m2 = jnp.max(masked, axis=-1, keepdims=True)
    idx2 = jnp.min(jnp.where(masked == m2, iota, e), axis=-1, keepdims=True)

    z = jnp.sum(jnp.exp(logits - m1), axis=-1, keepdims=True)
    s1 = 1.0 / z
    s2 = jnp.exp(m2 - m1) / z
    denom = s1 + s2 + TOPK_EPS
    idx_ref[...] = jnp.concatenate([idx1, idx2], axis=1)
    w_ref[...] = jnp.concatenate([s1 / denom, s2 / denom], axis=1)


@functools.partial(jax.jit, static_argnames=("interpret",))
def _gate(x, wt, interpret=False):
    n, h = x.shape
    e = wt.shape[1]
    grid = (n // BLOCK_M,)
    return pl.pallas_call(
        _gate_block,
        grid=grid,
        in_specs=[
            pl.BlockSpec((BLOCK_M, h), lambda i: (i, 0)),
            pl.BlockSpec((h, e), lambda i: (0, 0)),
        ],
        out_specs=[
            pl.BlockSpec((BLOCK_M, 2), lambda i: (i, 0)),
            pl.BlockSpec((BLOCK_M, 2), lambda i: (i, 0)),
        ],
        out_shape=[
            jax.ShapeDtypeStruct((n, 2), jnp.int32),
            jax.ShapeDtypeStruct((n, 2), jnp.float32),
        ],
        compiler_params=pltpu.CompilerParams(
            dimension_semantics=("arbitrary",),
        ),
        interpret=interpret,
    )(x, wt)


def kernel(hidden_states, weight):
    bsz, seq_len, h = hidden_states.shape
    x = hidden_states.reshape(-1, h)
    topk_idx, topk_weight = _gate(x, weight.T)
    return (
        topk_idx.reshape(bsz, seq_len, -1),
        topk_weight.reshape(bsz, seq_len, -1),
    )
